# Initial kernel scaffold; baseline (speedup 1.0000x reference)
import jax
import jax.numpy as jnp
from jax.experimental import pallas as pl


def kernel(user, item, W_user_gmf, W_item_gmf, W_user_mlp, W_item_mlp, mlp_W0, mlp_b0, mlp_W1, mlp_b1, mlp_W2, mlp_b2, pred_W, pred_b):
    raise NotImplementedError("write your pallas kernel here")


# speedup vs baseline: 1.0454x; 1.0454x over previous
import jax
import jax.numpy as jnp
from jax.experimental import pallas as pl
from jax.experimental.pallas import tpu as pltpu
from jax.experimental.pallas import tpu_sc as plsc

BATCH = 16384
FACTOR = 32
MLP_DIM = 128
PACK = 128 // FACTOR
GATHER_WINDOW = 128


def _sc_gather_all(u4, i4, user, item, Wug128, Wig128, W_user_mlp, W_item_mlp):
    vector_mesh = plsc.VectorSubcoreMesh(
        core_axis_name="core", subcore_axis_name="subcore"
    )

    out_types = (
        jax.ShapeDtypeStruct((BATCH, 128), jnp.float32),
        jax.ShapeDtypeStruct((BATCH, 128), jnp.float32),
        jax.ShapeDtypeStruct((BATCH, MLP_DIM), jnp.float32),
        jax.ShapeDtypeStruct((BATCH, MLP_DIM), jnp.float32),
    )

    @pl.kernel(out_type=out_types, mesh=vector_mesh, scratch_types=[])
    def gather_kernel(u4_hbm, i4_hbm, u_hbm, i_hbm,
                      wug_hbm, wig_hbm, wum_hbm, wim_hbm,
                      gug_hbm, gig_hbm, eum_hbm, eim_hbm):
        def make_body(table_hbm):
            def body(idx_vmem, out_vmem):
                pltpu.sync_copy(table_hbm.at[idx_vmem.at[0]], out_vmem)
            return body

        def run(table_hbm, idx_hbm, out_hbm):
            pltpu.emit_pipeline(
                make_body(table_hbm),
                grid=(BATCH // GATHER_WINDOW,),
                in_specs=[pl.BlockSpec((1, GATHER_WINDOW),
                                       index_map=lambda i: (0, i))],
                out_specs=[pl.BlockSpec((GATHER_WINDOW, 128),
                                        index_map=lambda i: (i, 0))],
                core_axis_name=("core", "subcore"),
                dimension_semantics=(pltpu.PARALLEL,),
            )(idx_hbm, out_hbm)

        run(wug_hbm, u4_hbm, gug_hbm)
        run(wig_hbm, i4_hbm, gig_hbm)
        run(wum_hbm, u_hbm, eum_hbm)
        run(wim_hbm, i_hbm, eim_hbm)

    return gather_kernel(u4, i4, user, item, Wug128, Wig128,
                         W_user_mlp, W_item_mlp)


def _tc_dense_kernel(gug_ref, gig_ref, uhot_ref, ihot_ref, eum_ref, eim_ref,
                     w0a_ref, w0b_ref, b0_ref, w1_ref, b1_ref,
                     w2_ref, b2_ref, pwg_ref, pwm_ref, pb_ref, out_ref):
    eu_mlp = eum_ref[...]
    ei_mlp = eim_ref[...]
    h0 = jnp.dot(eu_mlp, w0a_ref[...], preferred_element_type=jnp.float32)
    h0 += jnp.dot(ei_mlp, w0b_ref[...], preferred_element_type=jnp.float32)
    h0 = jnp.maximum(h0 + b0_ref[...], 0.0)
    h1 = jnp.dot(h0, w1_ref[...], preferred_element_type=jnp.float32)
    h1 = jnp.maximum(h1 + b1_ref[...], 0.0)
    h2 = jnp.dot(h1, w2_ref[...], preferred_element_type=jnp.float32)
    h2 = jnp.maximum(h2 + b2_ref[...], 0.0)

    gug = gug_ref[...]
    gig = gig_ref[...]
    uhot = uhot_ref[...]
    ihot = ihot_ref[...]
    eu = gug[:, 0:FACTOR] * uhot[:, 0:1]
    ei = gig[:, 0:FACTOR] * ihot[:, 0:1]
    for c in range(1, PACK):
        eu += gug[:, c * FACTOR:(c + 1) * FACTOR] * uhot[:, c:c + 1]
        ei += gig[:, c * FACTOR:(c + 1) * FACTOR] * ihot[:, c:c + 1]
    gmf = eu * ei

    pred = jnp.sum(gmf * pwg_ref[...] + h2 * pwm_ref[...], axis=1)
    out_ref[...] = pred + pb_ref[0, 0]


def kernel(user, item, W_user_gmf, W_item_gmf, W_user_mlp, W_item_mlp,
           mlp_W0, mlp_b0, mlp_W1, mlp_b1, mlp_W2, mlp_b2, pred_W, pred_b):
    Wug128 = W_user_gmf.reshape(-1, 128)
    Wig128 = W_item_gmf.reshape(-1, 128)
    u4 = (user // PACK).reshape(1, BATCH).astype(jnp.int32)
    i4 = (item // PACK).reshape(1, BATCH).astype(jnp.int32)
    uhot = jax.nn.one_hot(user % PACK, PACK, dtype=jnp.float32)
    ihot = jax.nn.one_hot(item % PACK, PACK, dtype=jnp.float32)
    user2 = user.reshape(1, BATCH).astype(jnp.int32)
    item2 = item.reshape(1, BATCH).astype(jnp.int32)

    gug, gig, eu_mlp, ei_mlp = _sc_gather_all(
        u4, i4, user2, item2, Wug128, Wig128, W_user_mlp, W_item_mlp)

    w0a = mlp_W0[:, :MLP_DIM].T
    w0b = mlp_W0[:, MLP_DIM:].T
    w1 = mlp_W1.T
    w2 = mlp_W2.T
    b0 = mlp_b0.reshape(1, -1)
    b1 = mlp_b1.reshape(1, -1)
    b2 = mlp_b2.reshape(1, -1)
    pwg = pred_W[:, :FACTOR]
    pwm = pred_W[:, FACTOR:]
    pb = pred_b.reshape(1, 1)

    blk = 2048
    grid = (BATCH // blk,)
    row_spec = lambda d: pl.BlockSpec((blk, d), lambda i: (i, 0))
    full = lambda a: pl.BlockSpec(a.shape, lambda i: (0,) * a.ndim)

    out = pl.pallas_call(
        _tc_dense_kernel,
        grid=grid,
        in_specs=[
            row_spec(128), row_spec(128), row_spec(PACK), row_spec(PACK),
            row_spec(MLP_DIM), row_spec(MLP_DIM),
            full(w0a), full(w0b), full(b0),
            full(w1), full(b1), full(w2), full(b2),
            full(pwg), full(pwm), full(pb),
        ],
        out_specs=pl.BlockSpec((blk,), lambda i: (i,)),
        out_shape=jax.ShapeDtypeStruct((BATCH,), jnp.float32),
    )(gug, gig, uhot, ihot, eu_mlp, ei_mlp,
      w0a, w0b, b0, w1, b1, w2, b2, pwg, pwm, pb)
    return out


# speedup vs baseline: 1.2604x; 1.2057x over previous
import jax
import jax.numpy as jnp
from jax.experimental import pallas as pl
from jax.experimental.pallas import tpu as pltpu
from jax.experimental.pallas import tpu_sc as plsc

BATCH = 16384
FACTOR = 32
MLP_DIM = 128
GATHER_WINDOW = 128

_VECTOR_MESH = plsc.VectorSubcoreMesh(
    core_axis_name="core", subcore_axis_name="subcore"
)


def _gather_pipeline(table_hbm, idx_hbm, out_hbm):
    def body(idx_vmem, out_vmem):
        pltpu.sync_copy(table_hbm.at[idx_vmem.at[0]], out_vmem)

    pltpu.emit_pipeline(
        body,
        grid=(BATCH // GATHER_WINDOW,),
        in_specs=[pl.BlockSpec((1, GATHER_WINDOW), index_map=lambda i: (0, i))],
        out_specs=[pl.BlockSpec((GATHER_WINDOW, 128), index_map=lambda i: (i, 0))],
        core_axis_name=("core", "subcore"),
        dimension_semantics=(pltpu.PARALLEL,),
    )(idx_hbm, out_hbm)


def _sc_gather_mlp(user2, item2, W_user_mlp, W_item_mlp):
    out_types = (
        jax.ShapeDtypeStruct((BATCH, MLP_DIM), jnp.float32),
        jax.ShapeDtypeStruct((BATCH, MLP_DIM), jnp.float32),
    )

    @pl.kernel(out_type=out_types, mesh=_VECTOR_MESH, scratch_types=[])
    def gather_mlp(u_hbm, i_hbm, wum_hbm, wim_hbm, eum_hbm, eim_hbm):
        _gather_pipeline(wum_hbm, u_hbm, eum_hbm)
        _gather_pipeline(wim_hbm, i_hbm, eim_hbm)

    return gather_mlp(user2, item2, W_user_mlp, W_item_mlp)


def _sc_gather_gmf(user2, item2, Wug128, Wig128):
    out_types = (
        jax.ShapeDtypeStruct((BATCH, 128), jnp.float32),
        jax.ShapeDtypeStruct((BATCH, 128), jnp.float32),
    )

    @pl.kernel(out_type=out_types, mesh=_VECTOR_MESH, scratch_types=[])
    def gather_gmf(u_hbm, i_hbm, wug_hbm, wig_hbm, gug_hbm, gig_hbm):
        _gather_pipeline(wug_hbm, u_hbm, gug_hbm)
        _gather_pipeline(wig_hbm, i_hbm, gig_hbm)

    return gather_gmf(user2, item2, Wug128, Wig128)


def _tc_dense_kernel(gug_ref, gig_ref, eum_ref, eim_ref,
                     w0a_ref, w0b_ref, b0_ref, w1_ref, b1_ref,
                     w2_ref, b2_ref, pwg_ref, pwm_ref, pb_ref, out_ref):
    h0 = jnp.dot(eum_ref[...], w0a_ref[...], preferred_element_type=jnp.float32)
    h0 += jnp.dot(eim_ref[...], w0b_ref[...], preferred_element_type=jnp.float32)
    h0 = jnp.maximum(h0 + b0_ref[...], 0.0)
    h1 = jnp.dot(h0, w1_ref[...], preferred_element_type=jnp.float32)
    h1 = jnp.maximum(h1 + b1_ref[...], 0.0)
    h2 = jnp.dot(h1, w2_ref[...], preferred_element_type=jnp.float32)
    h2 = jnp.maximum(h2 + b2_ref[...], 0.0)
    g = gug_ref[...] * gig_ref[...]
    pred = jnp.dot(g, pwg_ref[...], preferred_element_type=jnp.float32)
    pred += jnp.dot(h2, pwm_ref[...], preferred_element_type=jnp.float32)
    out_ref[...] = pred + pb_ref[0, 0]


def kernel(user, item, W_user_gmf, W_item_gmf, W_user_mlp, W_item_mlp,
           mlp_W0, mlp_b0, mlp_W1, mlp_b1, mlp_W2, mlp_b2, pred_W, pred_b):
    user2 = user.reshape(1, BATCH).astype(jnp.int32)
    item2 = item.reshape(1, BATCH).astype(jnp.int32)

    Wug128 = jnp.pad(W_user_gmf, ((0, 0), (0, 128 - FACTOR)))
    Wig128 = jnp.pad(W_item_gmf, ((0, 0), (0, 128 - FACTOR)))

    eu_mlp, ei_mlp = _sc_gather_mlp(user2, item2, W_user_mlp, W_item_mlp)
    gug, gig = _sc_gather_gmf(user2, item2, Wug128, Wig128)

    w0a = mlp_W0[:, :MLP_DIM].T
    w0b = mlp_W0[:, MLP_DIM:].T
    w1 = mlp_W1.T
    w2 = mlp_W2.T
    b0 = mlp_b0.reshape(1, -1)
    b1 = mlp_b1.reshape(1, -1)
    b2 = mlp_b2.reshape(1, -1)
    pwg = jnp.pad(pred_W[:, :FACTOR], ((0, 0), (0, 128 - FACTOR))).T
    pwm = pred_W[:, FACTOR:].T
    pb = pred_b.reshape(1, 1)

    blk = 2048
    grid = (BATCH // blk,)
    row_spec = lambda d: pl.BlockSpec((blk, d), lambda i: (i, 0))
    full = lambda a: pl.BlockSpec(a.shape, lambda i: (0,) * a.ndim)

    out = pl.pallas_call(
        _tc_dense_kernel,
        grid=grid,
        in_specs=[
            row_spec(128), row_spec(128),
            row_spec(MLP_DIM), row_spec(MLP_DIM),
            full(w0a), full(w0b), full(b0),
            full(w1), full(b1), full(w2), full(b2),
            full(pwg), full(pwm), full(pb),
        ],
        out_specs=pl.BlockSpec((blk, 1), lambda i: (i, 0)),
        out_shape=jax.ShapeDtypeStruct((BATCH, 1), jnp.float32),
    )(gug, gig, eu_mlp, ei_mlp,
      w0a, w0b, b0, w1, b1, w2, b2, pwg, pwm, pb)
    return out.reshape(-1)


# speedup vs baseline: 1.3411x; 1.0640x over previous
import jax
import jax.numpy as jnp
from jax.experimental import pallas as pl
from jax.experimental.pallas import tpu as pltpu
from jax.experimental.pallas import tpu_sc as plsc

BATCH = 16384
FACTOR = 32
MLP_DIM = 128
GATHER_WINDOW = 256

_VECTOR_MESH = plsc.VectorSubcoreMesh(
    core_axis_name="core", subcore_axis_name="subcore"
)


def _gather_pipeline(table_hbm, idx_hbm, out_hbm):
    def body(idx_vmem, out_vmem):
        pltpu.sync_copy(table_hbm.at[idx_vmem.at[0]], out_vmem)

    pltpu.emit_pipeline(
        body,
        grid=(BATCH // GATHER_WINDOW,),
        in_specs=[pl.BlockSpec((1, GATHER_WINDOW), index_map=lambda i: (0, i))],
        out_specs=[pl.BlockSpec((GATHER_WINDOW, 128), index_map=lambda i: (i, 0))],
        core_axis_name=("core", "subcore"),
        dimension_semantics=(pltpu.PARALLEL,),
    )(idx_hbm, out_hbm)


def _sc_gather_mlp(user2, item2, W_user_mlp, W_item_mlp):
    out_types = (
        jax.ShapeDtypeStruct((BATCH, MLP_DIM), jnp.float32),
        jax.ShapeDtypeStruct((BATCH, MLP_DIM), jnp.float32),
    )

    @pl.kernel(out_type=out_types, mesh=_VECTOR_MESH, scratch_types=[])
    def gather_mlp(u_hbm, i_hbm, wum_hbm, wim_hbm, eum_hbm, eim_hbm):
        _gather_pipeline(wum_hbm, u_hbm, eum_hbm)
        _gather_pipeline(wim_hbm, i_hbm, eim_hbm)

    return gather_mlp(user2, item2, W_user_mlp, W_item_mlp)


def _sc_gather_gmf(user2, item2, Wboth):
    out_types = (
        jax.ShapeDtypeStruct((BATCH, 128), jnp.float32),
        jax.ShapeDtypeStruct((BATCH, 128), jnp.float32),
    )

    @pl.kernel(out_type=out_types, mesh=_VECTOR_MESH, scratch_types=[])
    def gather_gmf(u_hbm, i_hbm, wb_hbm, gu_hbm, gi_hbm):
        _gather_pipeline(wb_hbm, u_hbm, gu_hbm)
        _gather_pipeline(wb_hbm, i_hbm, gi_hbm)

    return gather_gmf(user2, item2, Wboth)


def _tc_dense_kernel(gug_ref, gig_ref, eum_ref, eim_ref,
                     w0a_ref, w0b_ref, b0_ref, w1_ref, b1_ref,
                     w2_ref, b2_ref, pwg_ref, pwm_ref, pb_ref, out_ref):
    h0 = jnp.dot(eum_ref[...], w0a_ref[...], preferred_element_type=jnp.float32)
    h0 += jnp.dot(eim_ref[...], w0b_ref[...], preferred_element_type=jnp.float32)
    h0 = jnp.maximum(h0 + b0_ref[...], 0.0)
    h1 = jnp.dot(h0, w1_ref[...], preferred_element_type=jnp.float32)
    h1 = jnp.maximum(h1 + b1_ref[...], 0.0)
    h2 = jnp.dot(h1, w2_ref[...], preferred_element_type=jnp.float32)
    h2 = jnp.maximum(h2 + b2_ref[...], 0.0)
    g = gug_ref[...] * jnp.roll(gig_ref[...], -FACTOR, axis=1)
    pred = jnp.dot(g, pwg_ref[...], preferred_element_type=jnp.float32)
    pred += jnp.dot(h2, pwm_ref[...], preferred_element_type=jnp.float32)
    out_ref[...] = pred + pb_ref[0, 0]


def kernel(user, item, W_user_gmf, W_item_gmf, W_user_mlp, W_item_mlp,
           mlp_W0, mlp_b0, mlp_W1, mlp_b1, mlp_W2, mlp_b2, pred_W, pred_b):
    user2 = user.reshape(1, BATCH).astype(jnp.int32)
    item2 = item.reshape(1, BATCH).astype(jnp.int32)

    Wboth = jnp.pad(jnp.concatenate([W_user_gmf, W_item_gmf], axis=1),
                    ((0, 0), (0, 128 - 2 * FACTOR)))

    eu_mlp, ei_mlp = _sc_gather_mlp(user2, item2, W_user_mlp, W_item_mlp)
    gug, gig = _sc_gather_gmf(user2, item2, Wboth)

    w0a = mlp_W0[:, :MLP_DIM].T
    w0b = mlp_W0[:, MLP_DIM:].T
    w1 = mlp_W1.T
    w2 = mlp_W2.T
    b0 = mlp_b0.reshape(1, -1)
    b1 = mlp_b1.reshape(1, -1)
    b2 = mlp_b2.reshape(1, -1)
    pwg = jnp.pad(pred_W[:, :FACTOR], ((0, 0), (0, 128 - FACTOR))).T
    pwm = pred_W[:, FACTOR:].T
    pb = pred_b.reshape(1, 1)

    blk = 2048
    grid = (BATCH // blk,)
    row_spec = lambda d: pl.BlockSpec((blk, d), lambda i: (i, 0))
    full = lambda a: pl.BlockSpec(a.shape, lambda i: (0,) * a.ndim)

    out = pl.pallas_call(
        _tc_dense_kernel,
        grid=grid,
        in_specs=[
            row_spec(128), row_spec(128),
            row_spec(MLP_DIM), row_spec(MLP_DIM),
            full(w0a), full(w0b), full(b0),
            full(w1), full(b1), full(w2), full(b2),
            full(pwg), full(pwm), full(pb),
        ],
        out_specs=pl.BlockSpec((blk, 1), lambda i: (i, 0)),
        out_shape=jax.ShapeDtypeStruct((BATCH, 1), jnp.float32),
    )(gug, gig, eu_mlp, ei_mlp,
      w0a, w0b, b0, w1, b1, w2, b2, pwg, pwm, pb)
    return out.reshape(-1)
